# baseline (speedup 1.0000x reference)
import functools

import jax
import jax.numpy as jnp
from jax import lax
from jax.experimental import pallas as pl
from jax.experimental.pallas import tpu as pltpu
from jax.experimental.pallas import tpu_sc as plsc

NUM_LAYERS = 3
CE = 128
CH_R = 128
LANES = 16
G = 16
B = 5
LAG = 3


def _fill_const(buf, rows, width, value):
    vec = jnp.full((LANES,), value, dtype=jnp.float32)

    def body(i, carry):
        for h in range(0, width, LANES):
            buf[i, pl.ds(h, LANES)] = vec
        return carry

    lax.fori_loop(0, rows, body, 0)


def _rsqrt16(d):
    i = lax.bitcast_convert_type(d, jnp.int32)
    i = jnp.int32(0x5F3759DF) - (i >> 1)
    y = lax.bitcast_convert_type(i, jnp.float32)
    for _ in range(3):
        y = y * (1.5 - 0.5 * d * y * y)
    return jnp.where(d > 0.5, y, 0.0)


def _sc_body(N, N_pad, R, NCH, H, NS, e3_hbm, x0_hbm, out_hbm, acc_hbm,
             z_hbm, accum, idx_s, idx_d, v0, v1, v2, v3, v4, dinv_v,
             gs0, gs1, gs2, gs3, gs4, ss0, ss1, ss2, ss3, ss4):
    c = lax.axis_index("c")
    s = lax.axis_index("s")
    row0 = s * R
    c0 = (s * NCH) // NS
    c1 = ((s + 1) * NCH) // NS
    nsup = (c1 - c0) // G
    ntail = (c1 - c0) - nsup * G
    rows = [v0, v1, v2, v3, v4]
    gsem = [gs0, gs1, gs2, gs3, gs4]
    ssem = [ss0, ss1, ss2, ss3, ss4]
    src2d = e3_hbm.at[0]
    dst2d = e3_hbm.at[1]
    cH = c * H
    NCHUNK = R // CH_R

    def dv_splat(lrow):
        return plsc.load_gather(
            dinv_v, [jnp.full((LANES,), lrow, dtype=jnp.int32)])

    def zero_my_rows():
        _fill_const(v0, CH_R, H, 0.0)

        def body(k, carry):
            pltpu.sync_copy(v0, accum.at[pl.ds(row0 + k * CH_R, CH_R)])
            return carry
        lax.fori_loop(0, R // CH_R, body, 0)

    zero_my_rows()
    plsc.subcore_barrier()
    _fill_const(v0, CH_R, H, 1.0)

    def deg_super(sj, carry):
        pltpu.sync_copy(dst2d.at[pl.ds(c0 + sj * G, G)], idx_d)
        descs = []
        for k in range(G):
            descs.append(pltpu.async_copy(
                v0, accum.at[idx_d.at[k]], ssem[k % B], add=True))
            if k % B == B - 1 or k == G - 1:
                for dd in descs:
                    dd.wait()
                descs = []
        return carry
    lax.fori_loop(0, nsup, deg_super, 0)

    def deg_tail(t, carry):
        pltpu.sync_copy(dst2d.at[c0 + nsup * G + t], idx_d.at[0])
        pltpu.async_copy(v0, accum.at[idx_d.at[0]], ss0, add=True).wait()
        return carry
    lax.fori_loop(0, ntail, deg_tail, 0)
    plsc.subcore_barrier()

    def init_body(k, carry):
        start = row0 + k * CH_R
        lstart = k * CH_R
        pltpu.sync_copy(accum.at[pl.ds(start, CH_R)], v0)

        def grp(i, carry2):
            ridx = i * LANES + lax.iota(jnp.int32, 16)
            cidx = lax.iota(jnp.int32, 16)
            deg16 = plsc.load_gather(v0, [ridx, cidx])
            dinv_v[pl.ds(lstart + i * LANES, LANES)] = _rsqrt16(deg16)
            return carry2
        lax.fori_loop(0, CH_R // LANES, grp, 0)

        @pl.when(start + CH_R <= N)
        def _():
            pltpu.sync_copy(x0_hbm.at[pl.ds(start, CH_R), pl.ds(cH, H)], v1)

        @pl.when(jnp.logical_and(start < N, start + CH_R > N))
        def _():
            for sub in range(CH_R // 8):
                @pl.when(start + (sub + 1) * 8 <= N)
                def _():
                    pltpu.sync_copy(
                        x0_hbm.at[pl.ds(start + sub * 8, 8), pl.ds(cH, H)],
                        v1.at[pl.ds(sub * 8, 8)])

        def rowf(i, carry2):
            dv = dv_splat(lstart + i)
            for h in range(0, H, LANES):
                x0v = v1[i, pl.ds(h, LANES)]
                v0[i, pl.ds(h, LANES)] = dv * x0v
                v1[i, pl.ds(h, LANES)] = 0.25 * x0v
            return carry2
        lax.fori_loop(0, CH_R, rowf, 0)
        pltpu.sync_copy(v0, z_hbm.at[c].at[pl.ds(start, CH_R)])
        pltpu.sync_copy(v1, acc_hbm.at[c].at[pl.ds(start, CH_R)])
        return carry
    lax.fori_loop(0, R // CH_R, init_body, 0)

    for layer in range(NUM_LAYERS):
        last = layer == NUM_LAYERS - 1
        zero_my_rows()
        plsc.subcore_barrier()

        def edge_super(sj, carry):
            base = c0 + sj * G
            pltpu.sync_copy(src2d.at[pl.ds(base, G)], idx_s)
            pltpu.sync_copy(dst2d.at[pl.ds(base, G)], idx_d)
            dg = [None] * B
            dsc = [None] * B
            for b in range(B):
                dg[b] = pltpu.async_copy(z_hbm.at[c].at[idx_s.at[b]],
                                         rows[b], gsem[b])
            for k in range(G):
                b = k % B
                j = k - LAG
                if j >= 0 and j + B < G:
                    bp = j % B
                    dsc[bp].wait()
                    dg[bp] = pltpu.async_copy(
                        z_hbm.at[c].at[idx_s.at[j + B]], rows[bp], gsem[bp])
                dg[b].wait()
                dsc[b] = pltpu.async_copy(rows[b], accum.at[idx_d.at[k]],
                                          ssem[b], add=True)
            for k in range(G - B, G):
                dsc[k % B].wait()
            return carry
        lax.fori_loop(0, nsup, edge_super, 0)

        def edge_tail(t, carry):
            ci = c0 + nsup * G + t
            pltpu.sync_copy(src2d.at[ci], idx_s.at[0])
            pltpu.sync_copy(dst2d.at[ci], idx_d.at[0])
            pltpu.async_copy(z_hbm.at[c].at[idx_s.at[0]], rows[0],
                             gs0).wait()
            pltpu.async_copy(rows[0], accum.at[idx_d.at[0]], ss0,
                             add=True).wait()
            return carry
        lax.fori_loop(0, ntail, edge_tail, 0)
        plsc.subcore_barrier()

        def start_chunk(k, wb, ob, sem_a, sem_b):
            start = row0 + k * CH_R
            ra = pltpu.async_copy(accum.at[pl.ds(start, CH_R)], wb, sem_a)
            rb = pltpu.async_copy(acc_hbm.at[c].at[pl.ds(start, CH_R)], ob,
                                  sem_b)
            return start, ra, rb

        def finish_chunk(start, lstart, wb, ob, sem_a, sem_b):
            def rowf(i, carry2):
                for r in (2 * i, 2 * i + 1):
                    dv = dv_splat(lstart + r)
                    lo = wb[r, pl.ds(0, LANES)] * dv
                    hi = wb[r, pl.ds(16, LANES)] * dv
                    ob[r, pl.ds(0, LANES)] = ob[r, pl.ds(0, LANES)] + 0.25 * lo
                    ob[r, pl.ds(16, LANES)] = (ob[r, pl.ds(16, LANES)]
                                               + 0.25 * hi)
                    if not last:
                        wb[r, pl.ds(0, LANES)] = lo * dv
                        wb[r, pl.ds(16, LANES)] = hi * dv
                return carry2
            lax.fori_loop(0, CH_R // 2, rowf, 0)
            if last:
                wa = pltpu.async_copy(
                    ob, out_hbm.at[pl.ds(start, CH_R), pl.ds(cH, H)], sem_b)
                return (wa,)
            wa = pltpu.async_copy(ob, acc_hbm.at[c].at[pl.ds(start, CH_R)],
                                  sem_b)
            wz = pltpu.async_copy(wb, z_hbm.at[c].at[pl.ds(start, CH_R)],
                                  sem_a)
            return (wa, wz)

        def scale_pair(p, carry):
            k0 = 2 * p
            k1 = 2 * p + 1
            st0, ra0, rb0 = start_chunk(k0, v0, v1, gs0, gs1)
            st1, ra1, rb1 = start_chunk(k1, v2, v3, gs2, gs3)
            ra0.wait()
            rb0.wait()
            ws0 = finish_chunk(st0, k0 * CH_R, v0, v1, gs0, gs1)
            ra1.wait()
            rb1.wait()
            ws1 = finish_chunk(st1, k1 * CH_R, v2, v3, gs2, gs3)
            for w in ws0 + ws1:
                w.wait()
            return carry
        lax.fori_loop(0, NCHUNK // 2, scale_pair, 0)
        for k in range(NCHUNK - (NCHUNK % 2), NCHUNK):
            st0, ra0, rb0 = start_chunk(k, v0, v1, gs0, gs1)
            ra0.wait()
            rb0.wait()
            for w in finish_chunk(st0, k * CH_R, v0, v1, gs0, gs1):
                w.wait()


def kernel(edge_index, user_emb, item_emb):
    U = user_emb.shape[0]
    N = U + item_emb.shape[0]
    D = user_emb.shape[1]
    E = edge_index.shape[1]
    assert E % CE == 0
    NCH = E // CE

    info = plsc.get_sparse_core_info()
    NC, NS = info.num_cores, info.num_subcores
    H = D // NC

    R = -(-N // (NS * CH_R)) * CH_R
    N_pad = NS * R

    e3 = edge_index.reshape(2, NCH, CE)
    x0 = jnp.concatenate([user_emb, item_emb], axis=0)

    mesh = plsc.VectorSubcoreMesh(core_axis_name="c", subcore_axis_name="s")
    body = functools.partial(_sc_body, N, N_pad, R, NCH, H, NS)
    out, _acc, _z = pl.kernel(
        body,
        out_type=(jax.ShapeDtypeStruct((N_pad, D), jnp.float32),
                  jax.ShapeDtypeStruct((NC, N_pad, H), jnp.float32),
                  jax.ShapeDtypeStruct((NC, N_pad, H), jnp.float32)),
        mesh=mesh,
        compiler_params=pltpu.CompilerParams(
            use_tc_tiling_on_sc=False, needs_layout_passes=False),
        scratch_types=(
            [pltpu.VMEM_SHARED((N_pad, H), jnp.float32)]
            + [pltpu.VMEM((G, CE), jnp.int32)] * 2
            + [pltpu.VMEM((CE, H), jnp.float32)] * B
            + [pltpu.VMEM((R,), jnp.float32)]
            + [pltpu.SemaphoreType.DMA] * (2 * B)
        ),
    )(e3, x0)

    return out[:U], out[U:N]


# speedup vs baseline: 1.0548x; 1.0548x over previous
import functools

import jax
import jax.numpy as jnp
from jax import lax
from jax.experimental import pallas as pl
from jax.experimental.pallas import tpu as pltpu
from jax.experimental.pallas import tpu_sc as plsc

NUM_LAYERS = 3
CE = 128
CH_R = 128
LANES = 16
G = 16
B = 5
LAG = 1


def _fill_const(buf, rows, width, value):
    vec = jnp.full((LANES,), value, dtype=jnp.float32)

    def body(i, carry):
        for h in range(0, width, LANES):
            buf[i, pl.ds(h, LANES)] = vec
        return carry

    lax.fori_loop(0, rows, body, 0)


def _rsqrt16(d):
    i = lax.bitcast_convert_type(d, jnp.int32)
    i = jnp.int32(0x5F3759DF) - (i >> 1)
    y = lax.bitcast_convert_type(i, jnp.float32)
    for _ in range(3):
        y = y * (1.5 - 0.5 * d * y * y)
    return jnp.where(d > 0.5, y, 0.0)


def _sc_body(N, N_pad, R, NCH, H, NS, e3_hbm, x0_hbm, out_hbm, acc_hbm,
             z_hbm, accum, idx_s, idx_d, v0, v1, v2, v3, v4, dinv_v,
             gs0, gs1, gs2, gs3, gs4, ss0, ss1, ss2, ss3, ss4):
    c = lax.axis_index("c")
    s = lax.axis_index("s")
    row0 = s * R
    c0 = (s * NCH) // NS
    c1 = ((s + 1) * NCH) // NS
    nsup = (c1 - c0) // G
    ntail = (c1 - c0) - nsup * G
    rows = [v0, v1, v2, v3, v4]
    gsem = [gs0, gs1, gs2, gs3, gs4]
    ssem = [ss0, ss1, ss2, ss3, ss4]
    src2d = e3_hbm.at[0]
    dst2d = e3_hbm.at[1]
    cH = c * H
    NCHUNK = R // CH_R

    def dv_splat(lrow):
        return plsc.load_gather(
            dinv_v, [jnp.full((LANES,), lrow, dtype=jnp.int32)])

    def zero_my_rows():
        _fill_const(v0, CH_R, H, 0.0)

        def body(k, carry):
            pltpu.sync_copy(v0, accum.at[pl.ds(row0 + k * CH_R, CH_R)])
            return carry
        lax.fori_loop(0, R // CH_R, body, 0)

    zero_my_rows()
    plsc.subcore_barrier()
    _fill_const(v0, CH_R, H, 1.0)

    def deg_super(sj, carry):
        pltpu.sync_copy(dst2d.at[pl.ds(c0 + sj * G, G)], idx_d)
        descs = []
        for k in range(G):
            descs.append(pltpu.async_copy(
                v0, accum.at[idx_d.at[k]], ssem[k % B], add=True))
            if k % B == B - 1 or k == G - 1:
                for dd in descs:
                    dd.wait()
                descs = []
        return carry
    lax.fori_loop(0, nsup, deg_super, 0)

    def deg_tail(t, carry):
        pltpu.sync_copy(dst2d.at[c0 + nsup * G + t], idx_d.at[0])
        pltpu.async_copy(v0, accum.at[idx_d.at[0]], ss0, add=True).wait()
        return carry
    lax.fori_loop(0, ntail, deg_tail, 0)
    plsc.subcore_barrier()

    def init_body(k, carry):
        start = row0 + k * CH_R
        lstart = k * CH_R
        pltpu.sync_copy(accum.at[pl.ds(start, CH_R)], v0)

        def grp(i, carry2):
            ridx = i * LANES + lax.iota(jnp.int32, 16)
            cidx = lax.iota(jnp.int32, 16)
            deg16 = plsc.load_gather(v0, [ridx, cidx])
            dinv_v[pl.ds(lstart + i * LANES, LANES)] = _rsqrt16(deg16)
            return carry2
        lax.fori_loop(0, CH_R // LANES, grp, 0)

        @pl.when(start + CH_R <= N)
        def _():
            pltpu.sync_copy(x0_hbm.at[pl.ds(start, CH_R), pl.ds(cH, H)], v1)

        @pl.when(jnp.logical_and(start < N, start + CH_R > N))
        def _():
            for sub in range(CH_R // 8):
                @pl.when(start + (sub + 1) * 8 <= N)
                def _():
                    pltpu.sync_copy(
                        x0_hbm.at[pl.ds(start + sub * 8, 8), pl.ds(cH, H)],
                        v1.at[pl.ds(sub * 8, 8)])

        def rowf(i, carry2):
            dv = dv_splat(lstart + i)
            for h in range(0, H, LANES):
                x0v = v1[i, pl.ds(h, LANES)]
                v0[i, pl.ds(h, LANES)] = dv * x0v
                v1[i, pl.ds(h, LANES)] = 0.25 * x0v
            return carry2
        lax.fori_loop(0, CH_R, rowf, 0)
        pltpu.sync_copy(v0, z_hbm.at[c].at[pl.ds(start, CH_R)])
        pltpu.sync_copy(v1, acc_hbm.at[c].at[pl.ds(start, CH_R)])
        return carry
    lax.fori_loop(0, R // CH_R, init_body, 0)

    for layer in range(NUM_LAYERS):
        last = layer == NUM_LAYERS - 1
        zero_my_rows()
        plsc.subcore_barrier()

        def edge_super(sj, carry):
            base = c0 + sj * G
            pltpu.sync_copy(src2d.at[pl.ds(base, G)], idx_s)
            pltpu.sync_copy(dst2d.at[pl.ds(base, G)], idx_d)
            dg = [None] * B
            dsc = [None] * B
            for b in range(B):
                dg[b] = pltpu.async_copy(z_hbm.at[c].at[idx_s.at[b]],
                                         rows[b], gsem[b])
            for k in range(G):
                b = k % B
                j = k - LAG
                if j >= 0 and j + B < G:
                    bp = j % B
                    dsc[bp].wait()
                    dg[bp] = pltpu.async_copy(
                        z_hbm.at[c].at[idx_s.at[j + B]], rows[bp], gsem[bp])
                dg[b].wait()
                dsc[b] = pltpu.async_copy(rows[b], accum.at[idx_d.at[k]],
                                          ssem[b], add=True)
            for k in range(G - B, G):
                dsc[k % B].wait()
            return carry
        lax.fori_loop(0, nsup, edge_super, 0)

        def edge_tail(t, carry):
            ci = c0 + nsup * G + t
            pltpu.sync_copy(src2d.at[ci], idx_s.at[0])
            pltpu.sync_copy(dst2d.at[ci], idx_d.at[0])
            pltpu.async_copy(z_hbm.at[c].at[idx_s.at[0]], rows[0],
                             gs0).wait()
            pltpu.async_copy(rows[0], accum.at[idx_d.at[0]], ss0,
                             add=True).wait()
            return carry
        lax.fori_loop(0, ntail, edge_tail, 0)
        plsc.subcore_barrier()

        def start_chunk(k, wb, ob, sem_a, sem_b):
            start = row0 + k * CH_R
            ra = pltpu.async_copy(accum.at[pl.ds(start, CH_R)], wb, sem_a)
            rb = pltpu.async_copy(acc_hbm.at[c].at[pl.ds(start, CH_R)], ob,
                                  sem_b)
            return start, ra, rb

        def finish_chunk(start, lstart, wb, ob, sem_a, sem_b):
            def rowf(i, carry2):
                for r in (2 * i, 2 * i + 1):
                    dv = dv_splat(lstart + r)
                    lo = wb[r, pl.ds(0, LANES)] * dv
                    hi = wb[r, pl.ds(16, LANES)] * dv
                    ob[r, pl.ds(0, LANES)] = ob[r, pl.ds(0, LANES)] + 0.25 * lo
                    ob[r, pl.ds(16, LANES)] = (ob[r, pl.ds(16, LANES)]
                                               + 0.25 * hi)
                    if not last:
                        wb[r, pl.ds(0, LANES)] = lo * dv
                        wb[r, pl.ds(16, LANES)] = hi * dv
                return carry2
            lax.fori_loop(0, CH_R // 2, rowf, 0)
            if last:
                wa = pltpu.async_copy(
                    ob, out_hbm.at[pl.ds(start, CH_R), pl.ds(cH, H)], sem_b)
                return (wa,)
            wa = pltpu.async_copy(ob, acc_hbm.at[c].at[pl.ds(start, CH_R)],
                                  sem_b)
            wz = pltpu.async_copy(wb, z_hbm.at[c].at[pl.ds(start, CH_R)],
                                  sem_a)
            return (wa, wz)

        def scale_pair(p, carry):
            k0 = 2 * p
            k1 = 2 * p + 1
            st0, ra0, rb0 = start_chunk(k0, v0, v1, gs0, gs1)
            st1, ra1, rb1 = start_chunk(k1, v2, v3, gs2, gs3)
            ra0.wait()
            rb0.wait()
            ws0 = finish_chunk(st0, k0 * CH_R, v0, v1, gs0, gs1)
            ra1.wait()
            rb1.wait()
            ws1 = finish_chunk(st1, k1 * CH_R, v2, v3, gs2, gs3)
            for w in ws0 + ws1:
                w.wait()
            return carry
        lax.fori_loop(0, NCHUNK // 2, scale_pair, 0)
        for k in range(NCHUNK - (NCHUNK % 2), NCHUNK):
            st0, ra0, rb0 = start_chunk(k, v0, v1, gs0, gs1)
            ra0.wait()
            rb0.wait()
            for w in finish_chunk(st0, k * CH_R, v0, v1, gs0, gs1):
                w.wait()


def kernel(edge_index, user_emb, item_emb):
    U = user_emb.shape[0]
    N = U + item_emb.shape[0]
    D = user_emb.shape[1]
    E = edge_index.shape[1]
    assert E % CE == 0
    NCH = E // CE

    info = plsc.get_sparse_core_info()
    NC, NS = info.num_cores, info.num_subcores
    H = D // NC

    R = -(-N // (NS * CH_R)) * CH_R
    N_pad = NS * R

    e3 = edge_index.reshape(2, NCH, CE)
    x0 = jnp.concatenate([user_emb, item_emb], axis=0)

    mesh = plsc.VectorSubcoreMesh(core_axis_name="c", subcore_axis_name="s")
    body = functools.partial(_sc_body, N, N_pad, R, NCH, H, NS)
    out, _acc, _z = pl.kernel(
        body,
        out_type=(jax.ShapeDtypeStruct((N_pad, D), jnp.float32),
                  jax.ShapeDtypeStruct((NC, N_pad, H), jnp.float32),
                  jax.ShapeDtypeStruct((NC, N_pad, H), jnp.float32)),
        mesh=mesh,
        compiler_params=pltpu.CompilerParams(
            use_tc_tiling_on_sc=False, needs_layout_passes=False),
        scratch_types=(
            [pltpu.VMEM_SHARED((N_pad, H), jnp.float32)]
            + [pltpu.VMEM((G, CE), jnp.int32)] * 2
            + [pltpu.VMEM((CE, H), jnp.float32)] * B
            + [pltpu.VMEM((R,), jnp.float32)]
            + [pltpu.SemaphoreType.DMA] * (2 * B)
        ),
    )(e3, x0)

    return out[:U], out[U:N]
